# baseline (speedup 1.0000x reference)
import functools

import jax
import jax.numpy as jnp
from jax import lax
from jax.experimental import pallas as pl
from jax.experimental.pallas import tpu as pltpu
from jax.experimental.pallas import tpu_sc as plsc

VOCAB = 100000
EMBED_DIM = 128
BATCH = 4096
CTX = 20

_INFO = plsc.get_sparse_core_info()
_NC, _NS = _INFO.num_cores, _INFO.num_subcores
_NW = _NC * _NS
_ROWS_PER_W = BATCH // _NW
_ROWS_PER_CHUNK = 4
_IDX_PER_CHUNK = _ROWS_PER_CHUNK * CTX
_CHUNKS = _ROWS_PER_W // _ROWS_PER_CHUNK


def _emb_sum_body(x_hbm, table_hbm, out_hbm, idx_v, rows_v, acc_v, sem):
    wid = lax.axis_index("s") * _NC + lax.axis_index("c")
    base_row = wid * _ROWS_PER_W
    pltpu.sync_copy(x_hbm.at[pl.ds(base_row * CTX, _ROWS_PER_W * CTX)], idx_v)

    def chunk_body(c, carry):
        idx_slice = idx_v.at[pl.ds(c * _IDX_PER_CHUNK, _IDX_PER_CHUNK)]
        pltpu.async_copy(table_hbm.at[idx_slice], rows_v, sem).wait()
        for r in range(_ROWS_PER_CHUNK):
            row = c * _ROWS_PER_CHUNK + r
            for g in range(EMBED_DIM // 16):
                def add_one(j, acc, _r=r, _g=g):
                    return acc + rows_v[_r * CTX + j, pl.ds(_g * 16, 16)]
                acc = lax.fori_loop(0, CTX, add_one,
                                    jnp.zeros((16,), jnp.float32))
                acc_v[row, pl.ds(g * 16, 16)] = acc
        return carry

    lax.fori_loop(0, _CHUNKS, chunk_body, 0)
    pltpu.sync_copy(acc_v, out_hbm.at[pl.ds(base_row, _ROWS_PER_W)])


_emb_sum = functools.partial(
    pl.kernel,
    out_type=jax.ShapeDtypeStruct((BATCH, EMBED_DIM), jnp.float32),
    mesh=plsc.VectorSubcoreMesh(core_axis_name="c", subcore_axis_name="s"),
    scratch_types=[
        pltpu.VMEM((_ROWS_PER_W * CTX,), jnp.int32),
        pltpu.VMEM((_IDX_PER_CHUNK, EMBED_DIM), jnp.float32),
        pltpu.VMEM((_ROWS_PER_W, EMBED_DIM), jnp.float32),
        pltpu.SemaphoreType.DMA,
    ],
)(_emb_sum_body)


_V_BLK = 1024


def _proj_body(emb_ref, w_ref, b_ref, out_ref, ebf_ref):
    @pl.when(pl.program_id(0) == 0)
    def _cast_once():
        ebf_ref[...] = emb_ref[...].astype(jnp.bfloat16)

    out_ref[...] = jnp.broadcast_to(b_ref[...], out_ref.shape)


_B_BLK = 1024
_V_BLK2 = 4096


def _projection(embedded, W, b2d):
    nv = pl.cdiv(VOCAB, _V_BLK2)
    nb = BATCH // _B_BLK
    return pl.pallas_call(
        _proj_body,
        grid=(nb, nv),
        in_specs=[
            pl.BlockSpec((_B_BLK, EMBED_DIM), lambda i, v: (i, 0)),
            pl.BlockSpec((EMBED_DIM, _V_BLK2), lambda i, v: (0, v)),
            pl.BlockSpec((1, _V_BLK2), lambda i, v: (0, v)),
        ],
        out_specs=pl.BlockSpec((_B_BLK, _V_BLK2), lambda i, v: (i, v)),
        out_shape=jax.ShapeDtypeStruct((BATCH, VOCAB), jnp.float32),
        scratch_shapes=[pltpu.VMEM((_B_BLK, EMBED_DIM), jnp.bfloat16)],
    )(embedded, W, b2d)


def kernel(x, emb_table, W, b):
    x_flat = x.reshape(-1).astype(jnp.int32)
    embedded = _emb_sum(x_flat, emb_table)
    return _projection(embedded, W, b.reshape(1, VOCAB))


# speedup vs baseline: 1.0114x; 1.0114x over previous
import functools

import jax
import jax.numpy as jnp
from jax import lax
from jax.experimental import pallas as pl
from jax.experimental.pallas import tpu as pltpu
from jax.experimental.pallas import tpu_sc as plsc

VOCAB = 100000
EMBED_DIM = 128
BATCH = 4096
CTX = 20

_INFO = plsc.get_sparse_core_info()
_NC, _NS = _INFO.num_cores, _INFO.num_subcores
_NW = _NC * _NS
_ROWS_PER_W = BATCH // _NW
_ROWS_PER_CHUNK = 4
_IDX_PER_CHUNK = _ROWS_PER_CHUNK * CTX
_CHUNKS = _ROWS_PER_W // _ROWS_PER_CHUNK


def _emb_sum_body(x_hbm, table_hbm, out_hbm, idx_v, rows_v, acc_v, sem):
    wid = lax.axis_index("s") * _NC + lax.axis_index("c")
    base_row = wid * _ROWS_PER_W
    pltpu.sync_copy(x_hbm.at[pl.ds(base_row * CTX, _ROWS_PER_W * CTX)], idx_v)

    def chunk_body(c, carry):
        idx_slice = idx_v.at[pl.ds(c * _IDX_PER_CHUNK, _IDX_PER_CHUNK)]
        pltpu.async_copy(table_hbm.at[idx_slice], rows_v, sem).wait()
        for r in range(_ROWS_PER_CHUNK):
            row = c * _ROWS_PER_CHUNK + r
            for g in range(EMBED_DIM // 16):
                def add_one(j, acc, _r=r, _g=g):
                    return acc + rows_v[_r * CTX + j, pl.ds(_g * 16, 16)]
                acc = lax.fori_loop(0, CTX, add_one,
                                    jnp.zeros((16,), jnp.float32))
                acc_v[row, pl.ds(g * 16, 16)] = acc
        return carry

    lax.fori_loop(0, _CHUNKS, chunk_body, 0)
    pltpu.sync_copy(acc_v, out_hbm.at[pl.ds(base_row, _ROWS_PER_W)])


_emb_sum = functools.partial(
    pl.kernel,
    out_type=jax.ShapeDtypeStruct((BATCH, EMBED_DIM), jnp.float32),
    mesh=plsc.VectorSubcoreMesh(core_axis_name="c", subcore_axis_name="s"),
    scratch_types=[
        pltpu.VMEM((_ROWS_PER_W * CTX,), jnp.int32),
        pltpu.VMEM((_IDX_PER_CHUNK, EMBED_DIM), jnp.float32),
        pltpu.VMEM((_ROWS_PER_W, EMBED_DIM), jnp.float32),
        pltpu.SemaphoreType.DMA,
    ],
)(_emb_sum_body)


_V_BLK = 1024
_NV = pl.cdiv(VOCAB, _V_BLK)
_V_TAIL = 782 * 128 - (_NV - 1) * _V_BLK
_NDMA = 4
_RB = BATCH // _NDMA


def _proj_body(emb_ref, w_ref, b_ref, out_ref, buf_ref, ebf_ref, sems):
    v = pl.program_id(0)
    slot = v % 2

    @pl.when(v == 0)
    def _cast_once():
        ebf_ref[...] = emb_ref[...].astype(jnp.bfloat16)

    def full_copy(s, d, voff):
        return pltpu.make_async_copy(
            buf_ref.at[s, pl.ds(d * _RB, _RB), :],
            out_ref.at[pl.ds(d * _RB, _RB), pl.ds(voff, _V_BLK)],
            sems.at[s, d])

    def tail_copy(s, d, voff):
        return pltpu.make_async_copy(
            buf_ref.at[s, pl.ds(d * _RB, _RB), pl.ds(0, _V_TAIL)],
            out_ref.at[pl.ds(d * _RB, _RB), pl.ds(voff, _V_TAIL)],
            sems.at[s, d])

    @pl.when(v >= 2)
    def _drain():
        for d in range(_NDMA):
            full_copy(slot, d, 0).wait()

    w = w_ref[...].astype(jnp.bfloat16)
    acc = lax.dot_general(ebf_ref[...], w, (((1,), (0,)), ((), ())),
                          preferred_element_type=jnp.float32)
    buf_ref[slot, :, :] = acc + b_ref[...]

    voff = v * _V_BLK

    @pl.when(v < _NV - 1)
    def _fire_full():
        for d in range(_NDMA):
            full_copy(slot, d, voff).start()

    @pl.when(v == _NV - 1)
    def _fire_tail_and_drain_all():
        for d in range(_NDMA):
            tail_copy(slot, d, voff).start()
        for d in range(_NDMA):
            full_copy(1 - slot, d, 0).wait()
            tail_copy(slot, d, 0).wait()


def _projection(embedded, W, b2d):
    return pl.pallas_call(
        _proj_body,
        grid=(_NV,),
        in_specs=[
            pl.BlockSpec((BATCH, EMBED_DIM), lambda v: (0, 0)),
            pl.BlockSpec((EMBED_DIM, _V_BLK), lambda v: (0, v)),
            pl.BlockSpec((1, _V_BLK), lambda v: (0, v)),
        ],
        out_specs=pl.BlockSpec(memory_space=pl.ANY),
        out_shape=jax.ShapeDtypeStruct((BATCH, VOCAB), jnp.float32),
        scratch_shapes=[
            pltpu.VMEM((2, BATCH, _V_BLK), jnp.float32),
            pltpu.VMEM((BATCH, EMBED_DIM), jnp.bfloat16),
            pltpu.SemaphoreType.DMA((2, _NDMA)),
        ],
    )(embedded, W, b2d)


def kernel(x, emb_table, W, b):
    x_flat = x.reshape(-1).astype(jnp.int32)
    embedded = _emb_sum(x_flat, emb_table)
    return _projection(embedded, W, b.reshape(1, VOCAB))


# speedup vs baseline: 3.4253x; 3.3868x over previous
import functools

import jax
import jax.numpy as jnp
from jax import lax
from jax.experimental import pallas as pl
from jax.experimental.pallas import tpu as pltpu
from jax.experimental.pallas import tpu_sc as plsc

VOCAB = 100000
EMBED_DIM = 128
BATCH = 4096
CTX = 20

_INFO = plsc.get_sparse_core_info()
_NC, _NS = _INFO.num_cores, _INFO.num_subcores
_NW = _NC * _NS
_ROWS_PER_W = BATCH // _NW
_ROWS_PER_CHUNK = 4
_IDX_PER_CHUNK = _ROWS_PER_CHUNK * CTX
_CHUNKS = _ROWS_PER_W // _ROWS_PER_CHUNK


def _emb_sum_body(x_hbm, table_hbm, out_hbm, idx_v, rows_v, acc_v, sem):
    wid = lax.axis_index("s") * _NC + lax.axis_index("c")
    base_row = wid * _ROWS_PER_W
    pltpu.sync_copy(x_hbm.at[pl.ds(base_row * CTX, _ROWS_PER_W * CTX)], idx_v)

    def chunk_body(c, carry):
        idx_slice = idx_v.at[pl.ds(c * _IDX_PER_CHUNK, _IDX_PER_CHUNK)]
        pltpu.async_copy(table_hbm.at[idx_slice], rows_v, sem).wait()
        for r in range(_ROWS_PER_CHUNK):
            row = c * _ROWS_PER_CHUNK + r
            for g in range(EMBED_DIM // 16):
                def add_one(j, acc, _r=r, _g=g):
                    return acc + rows_v[_r * CTX + j, pl.ds(_g * 16, 16)]
                acc = lax.fori_loop(0, CTX, add_one,
                                    jnp.zeros((16,), jnp.float32))
                acc_v[row, pl.ds(g * 16, 16)] = acc
        return carry

    lax.fori_loop(0, _CHUNKS, chunk_body, 0)
    pltpu.sync_copy(acc_v, out_hbm.at[pl.ds(base_row, _ROWS_PER_W)])


_emb_sum = functools.partial(
    pl.kernel,
    out_type=jax.ShapeDtypeStruct((BATCH, EMBED_DIM), jnp.float32),
    mesh=plsc.VectorSubcoreMesh(core_axis_name="c", subcore_axis_name="s"),
    scratch_types=[
        pltpu.VMEM((_ROWS_PER_W * CTX,), jnp.int32),
        pltpu.VMEM((_IDX_PER_CHUNK, EMBED_DIM), jnp.float32),
        pltpu.VMEM((_ROWS_PER_W, EMBED_DIM), jnp.float32),
        pltpu.SemaphoreType.DMA,
    ],
)(_emb_sum_body)


_V_BLK = 1024
_NV = pl.cdiv(VOCAB, _V_BLK)


def _proj_body(emb_ref, wt_ref, b_ref, out_ref, ebf_ref):
    @pl.when(pl.program_id(0) == 0)
    def _cast_once():
        ebf_ref[...] = emb_ref[...].astype(jnp.bfloat16)

    wt = wt_ref[...].astype(jnp.bfloat16)
    acc = lax.dot_general(wt, ebf_ref[...], (((1,), (1,)), ((), ())),
                          preferred_element_type=jnp.float32)
    bt = jnp.transpose(b_ref[...], (1, 0))
    out_ref[...] = acc + bt


def _projection(embedded, WT, b2d):
    return pl.pallas_call(
        _proj_body,
        grid=(_NV,),
        in_specs=[
            pl.BlockSpec((BATCH, EMBED_DIM), lambda v: (0, 0)),
            pl.BlockSpec((_V_BLK, EMBED_DIM), lambda v: (v, 0)),
            pl.BlockSpec((1, _V_BLK), lambda v: (0, v)),
        ],
        out_specs=pl.BlockSpec((_V_BLK, BATCH), lambda v: (v, 0)),
        out_shape=jax.ShapeDtypeStruct((VOCAB, BATCH), jnp.float32),
        scratch_shapes=[pltpu.VMEM((BATCH, EMBED_DIM), jnp.bfloat16)],
    )(embedded, WT, b2d)


def kernel(x, emb_table, W, b):
    x_flat = x.reshape(-1).astype(jnp.int32)
    embedded = _emb_sum(x_flat, emb_table)
    out_t = _projection(embedded, W.T, b.reshape(1, VOCAB))
    return out_t.T


# speedup vs baseline: 3.4636x; 1.0112x over previous
import functools

import jax
import jax.numpy as jnp
from jax import lax
from jax.experimental import pallas as pl
from jax.experimental.pallas import tpu as pltpu
from jax.experimental.pallas import tpu_sc as plsc

VOCAB = 100000
EMBED_DIM = 128
BATCH = 4096
CTX = 20

_INFO = plsc.get_sparse_core_info()
_NC, _NS = _INFO.num_cores, _INFO.num_subcores
_NW = _NC * _NS
_ROWS_PER_W = BATCH // _NW
_ROWS_PER_CHUNK = 4
_IDX_PER_CHUNK = _ROWS_PER_CHUNK * CTX
_CHUNKS = _ROWS_PER_W // _ROWS_PER_CHUNK


def _tree_sum16(vals):
    while len(vals) > 1:
        vals = [a + b for a, b in zip(vals[::2], vals[1::2])] + (
            [vals[-1]] if len(vals) % 2 else [])
    return vals[0]


def _emb_sum_body(x_hbm, table_hbm, out_hbm, idx_v, rows_v, acc_v, sem):
    wid = lax.axis_index("s") * _NC + lax.axis_index("c")
    base_row = wid * _ROWS_PER_W
    pltpu.sync_copy(x_hbm.at[pl.ds(base_row * CTX, _ROWS_PER_W * CTX)], idx_v)

    def chunk_body(c, carry):
        idx_slice = idx_v.at[pl.ds(c * _IDX_PER_CHUNK, _IDX_PER_CHUNK)]
        pltpu.async_copy(table_hbm.at[idx_slice], rows_v, sem).wait()
        for r in range(_ROWS_PER_CHUNK):
            row = c * _ROWS_PER_CHUNK + r
            for g in range(EMBED_DIM // 16):
                vals = [rows_v[r * CTX + j, pl.ds(g * 16, 16)]
                        for j in range(CTX)]
                acc_v[row, pl.ds(g * 16, 16)] = _tree_sum16(vals)
        return carry

    lax.fori_loop(0, _CHUNKS, chunk_body, 0)
    pltpu.sync_copy(acc_v, out_hbm.at[pl.ds(base_row, _ROWS_PER_W)])


_emb_sum = functools.partial(
    pl.kernel,
    out_type=jax.ShapeDtypeStruct((BATCH, EMBED_DIM), jnp.float32),
    mesh=plsc.VectorSubcoreMesh(core_axis_name="c", subcore_axis_name="s"),
    scratch_types=[
        pltpu.VMEM((_ROWS_PER_W * CTX,), jnp.int32),
        pltpu.VMEM((_IDX_PER_CHUNK, EMBED_DIM), jnp.float32),
        pltpu.VMEM((_ROWS_PER_W, EMBED_DIM), jnp.float32),
        pltpu.SemaphoreType.DMA,
    ],
)(_emb_sum_body)


_V_BLK = 1024
_NV = pl.cdiv(VOCAB, _V_BLK)


def _proj_body(emb_ref, wt_ref, b_ref, out_ref, ebf_ref):
    @pl.when(pl.program_id(0) == 0)
    def _cast_once():
        ebf_ref[...] = emb_ref[...].astype(jnp.bfloat16)

    acc = lax.dot_general(wt_ref[...], ebf_ref[...], (((1,), (1,)), ((), ())),
                          preferred_element_type=jnp.float32)
    bt = jnp.transpose(b_ref[...], (1, 0))
    out_ref[...] = acc + bt


def _projection(embedded, WT, b2d):
    return pl.pallas_call(
        _proj_body,
        grid=(_NV,),
        in_specs=[
            pl.BlockSpec((BATCH, EMBED_DIM), lambda v: (0, 0)),
            pl.BlockSpec((_V_BLK, EMBED_DIM), lambda v: (v, 0)),
            pl.BlockSpec((1, _V_BLK), lambda v: (0, v)),
        ],
        out_specs=pl.BlockSpec((_V_BLK, BATCH), lambda v: (v, 0)),
        out_shape=jax.ShapeDtypeStruct((VOCAB, BATCH), jnp.float32),
        scratch_shapes=[pltpu.VMEM((BATCH, EMBED_DIM), jnp.bfloat16)],
    )(embedded, WT, b2d)


def kernel(x, emb_table, W, b):
    x_flat = x.reshape(-1).astype(jnp.int32)
    wt_bf = W.T.astype(jnp.bfloat16)
    embedded = _emb_sum(x_flat, emb_table)
    out_t = _projection(embedded, wt_bf, b.reshape(1, VOCAB))
    return out_t.T
